# baseline (speedup 1.0000x reference)
import jax
import jax.numpy as jnp
from jax import lax
from jax.experimental import pallas as pl
from jax.experimental.pallas import tpu as pltpu
from jax.experimental.pallas import tpu_sc as plsc

N = 10000
E = 320000
D = 128
EPS = 1e-5

NC = 2
NS = 16
NW = NC * NS
E_PER = E // NW
CH1 = 80
NCH1 = E_PER // CH1
NLOC = 320
CH2 = 8000
NCH2 = E // CH2
SELC = ((CH2 + 16 + 127) // 128) * 128
BE = 2000
NBE = E // BE


def _bn_train(x, g, b):
    m = jnp.mean(x, axis=0, keepdims=True)
    v = jnp.mean((x - m) * (x - m), axis=0, keepdims=True)
    return (x - m) / jnp.sqrt(v + EPS) * g + b


def _tc_pre_body(f_ref, c_ref, wa1t, ba1, ga1, bta1, wa2t, ba2, ga2, bta2,
                 w1at, w1bt, b1, p_ref, q_ref):
    f = f_ref[...]
    c = c_ref[...]
    x = jnp.maximum(jnp.dot(f, wa1t[...], preferred_element_type=jnp.float32)
                    + ba1[...], 0.0)
    x = _bn_train(x, ga1[...], bta1[...])
    x = jnp.maximum(jnp.dot(x, wa2t[...], preferred_element_type=jnp.float32)
                    + ba2[...], 0.0)
    off = _bn_train(x, ga2[...], bta2[...])
    c2 = c + off
    p_ref[...] = (jnp.dot(f, w1at[...], preferred_element_type=jnp.float32)
                  + jnp.dot(c, w1bt[...], preferred_element_type=jnp.float32)
                  + b1[...])
    q_ref[...] = jnp.dot(c2, w1bt[...], preferred_element_type=jnp.float32)


def _tc_pre(f, c, wa1t, ba1, ga1, bta1, wa2t, ba2, ga2, bta2, w1at, w1bt, b1):
    return pl.pallas_call(
        _tc_pre_body,
        out_shape=[jax.ShapeDtypeStruct((N, D), jnp.float32),
                   jax.ShapeDtypeStruct((N, D), jnp.float32)],
    )(f, c, wa1t, ba1, ga1, bta1, wa2t, ba2, ga2, bta2, w1at, w1bt, b1)


def _sc1_body(src_hbm, dst_hbm, p_hbm, q_hbm, h_hbm, sh_hbm, sq_hbm,
              srcall, dstall, pv0, qv0, hv0, pv1, qv1, hv1, shv, sqv,
              sem_p0, sem_q0, sem_h0, sem_p1, sem_q1, sem_h1, sem_i):
    wid = lax.axis_index("s") * NC + lax.axis_index("c")
    base0 = wid * E_PER
    zero = jnp.zeros((16,), jnp.float32)
    init = (tuple(zero for _ in range(8)), tuple(zero for _ in range(8)))

    cp_s = pltpu.async_copy(src_hbm.at[pl.ds(base0, E_PER)], srcall, sem_i)
    cp_d = pltpu.async_copy(dst_hbm.at[pl.ds(base0, E_PER)], dstall, sem_i)
    cp_s.wait()
    cp_d.wait()

    def gathers(ci, pv, qv, sp, sq_):
        sl = pl.ds(ci * CH1, CH1)
        pltpu.async_copy(p_hbm.at[srcall.at[sl]], pv, sp)
        pltpu.async_copy(q_hbm.at[dstall.at[sl]], qv, sq_)

    def compute(ci, pv, qv, hv, carry):
        def row(i, cr):
            sh, sq = cr
            nsh = []
            nsq = []
            for k in range(8):
                sl = pl.ds(16 * k, 16)
                h = jnp.maximum(pv[i, sl] - qv[i, sl], 0.0)
                hv[i, sl] = h
                nsh.append(sh[k] + h)
                nsq.append(sq[k] + h * h)
            return (tuple(nsh), tuple(nsq))

        return lax.fori_loop(0, CH1, row, carry)

    def wait_g(pv, qv, sp, sq_):
        pltpu.make_async_copy(p_hbm.at[pl.ds(0, CH1)], pv, sp).wait()
        pltpu.make_async_copy(q_hbm.at[pl.ds(0, CH1)], qv, sq_).wait()

    def wait_h(hv, sh_):
        pltpu.make_async_copy(hv, h_hbm.at[pl.ds(0, CH1)], sh_).wait()

    gathers(0, pv0, qv0, sem_p0, sem_q0)

    def pair(i, carry):
        a = 2 * i
        gathers(a + 1, pv1, qv1, sem_p1, sem_q1)
        wait_g(pv0, qv0, sem_p0, sem_q0)

        @pl.when(i > 0)
        def _():
            wait_h(hv0, sem_h0)

        carry = compute(a, pv0, qv0, hv0, carry)
        pltpu.async_copy(hv0, h_hbm.at[pl.ds(base0 + a * CH1, CH1)], sem_h0)
        gathers(a + 2, pv0, qv0, sem_p0, sem_q0)
        wait_g(pv1, qv1, sem_p1, sem_q1)

        @pl.when(i > 0)
        def _():
            wait_h(hv1, sem_h1)

        carry = compute(a + 1, pv1, qv1, hv1, carry)
        pltpu.async_copy(hv1, h_hbm.at[pl.ds(base0 + (a + 1) * CH1, CH1)],
                         sem_h1)
        return carry

    carry = lax.fori_loop(0, (NCH1 - 1) // 2, pair, init)
    wait_g(pv0, qv0, sem_p0, sem_q0)
    wait_h(hv0, sem_h0)
    sh, sq = compute(NCH1 - 1, pv0, qv0, hv0, carry)
    pltpu.sync_copy(hv0, h_hbm.at[pl.ds(base0 + (NCH1 - 1) * CH1, CH1)])
    wait_h(hv1, sem_h1)

    for k in range(8):
        sl = pl.ds(16 * k, 16)
        shv[0, sl] = sh[k]
        sqv[0, sl] = sq[k]
    pltpu.sync_copy(shv, sh_hbm.at[pl.ds(wid, 1)])
    pltpu.sync_copy(sqv, sq_hbm.at[pl.ds(wid, 1)])


def _sc_pass1(src, dst, p, q):
    mesh = plsc.VectorSubcoreMesh(core_axis_name="c", subcore_axis_name="s")
    return pl.kernel(
        _sc1_body,
        out_type=[jax.ShapeDtypeStruct((E, D), jnp.float32),
                  jax.ShapeDtypeStruct((NW, D), jnp.float32),
                  jax.ShapeDtypeStruct((NW, D), jnp.float32)],
        mesh=mesh,
        compiler_params=pltpu.CompilerParams(needs_layout_passes=False),
        scratch_types=[pltpu.VMEM((E_PER,), jnp.int32),
                       pltpu.VMEM((E_PER,), jnp.int32),
                       pltpu.VMEM((CH1, D), jnp.float32),
                       pltpu.VMEM((CH1, D), jnp.float32),
                       pltpu.VMEM((CH1, D), jnp.float32),
                       pltpu.VMEM((CH1, D), jnp.float32),
                       pltpu.VMEM((CH1, D), jnp.float32),
                       pltpu.VMEM((CH1, D), jnp.float32),
                       pltpu.VMEM((1, D), jnp.float32),
                       pltpu.VMEM((1, D), jnp.float32),
                       pltpu.SemaphoreType.DMA,
                       pltpu.SemaphoreType.DMA,
                       pltpu.SemaphoreType.DMA,
                       pltpu.SemaphoreType.DMA,
                       pltpu.SemaphoreType.DMA,
                       pltpu.SemaphoreType.DMA,
                       pltpu.SemaphoreType.DMA],
    )(src, dst, p, q)


def _tc_z_body(h_ref, sh_ref, sq_ref, g1, bt1, w2t, b2, z_ref, sz_ref, szz_ref):
    i = pl.program_id(0)
    m1 = jnp.sum(sh_ref[...], axis=0, keepdims=True) * (1.0 / E)
    v1 = jnp.sum(sq_ref[...], axis=0, keepdims=True) * (1.0 / E) - m1 * m1
    inv1 = g1[...] / jnp.sqrt(v1 + EPS)
    hn = (h_ref[...] - m1) * inv1 + bt1[...]
    z = jnp.maximum(jnp.dot(hn, w2t[...], preferred_element_type=jnp.float32)
                    + b2[...], 0.0)
    z_ref[...] = z
    bs = jnp.sum(z, axis=0, keepdims=True)
    bss = jnp.sum(z * z, axis=0, keepdims=True)

    @pl.when(i == 0)
    def _():
        sz_ref[...] = bs
        szz_ref[...] = bss

    @pl.when(i > 0)
    def _():
        sz_ref[...] += bs
        szz_ref[...] += bss


def _tc_z(h, sh, sq, g1, bt1, w2t, b2):
    return pl.pallas_call(
        _tc_z_body,
        grid=(NBE,),
        in_specs=[pl.BlockSpec((BE, D), lambda i: (i, 0)),
                  pl.BlockSpec((NW, D), lambda i: (0, 0)),
                  pl.BlockSpec((NW, D), lambda i: (0, 0)),
                  pl.BlockSpec((1, D), lambda i: (0, 0)),
                  pl.BlockSpec((1, D), lambda i: (0, 0)),
                  pl.BlockSpec((D, D), lambda i: (0, 0)),
                  pl.BlockSpec((1, D), lambda i: (0, 0))],
        out_specs=[pl.BlockSpec((BE, D), lambda i: (i, 0)),
                   pl.BlockSpec((1, D), lambda i: (0, 0)),
                   pl.BlockSpec((1, D), lambda i: (0, 0))],
        out_shape=[jax.ShapeDtypeStruct((E, D), jnp.float32),
                   jax.ShapeDtypeStruct((1, D), jnp.float32),
                   jax.ShapeDtypeStruct((1, D), jnp.float32)],
    )(h, sh, sq, g1, bt1, w2t, b2)


def _sc2_body(dst_hbm, z_hbm, r_hbm, dstv0, dstv1, sele, seld, zb0, zb1, acc,
              sem_c0, sem_c1, sem_g0, sem_g1):
    wid = lax.axis_index("s") * NC + lax.axis_index("c")
    lo = wid * NLOC
    zero = jnp.zeros((16,), jnp.float32)
    izero = jnp.zeros((16,), jnp.int32)
    iota16 = lax.iota(jnp.int32, 16)

    def zrow(i, _):
        for k in range(8):
            acc[i, pl.ds(16 * k, 16)] = zero
        return 0

    lax.fori_loop(0, NLOC + 1, zrow, 0)

    def zsel(i, _):
        sele[pl.ds(i * 16, 16)] = izero
        return 0

    lax.fori_loop(0, SELC // 16, zsel, 0)

    def scan(dstv, cbase):
        def sg(g, cur):
            v = dstv[pl.ds(g * 16, 16)]
            dloc = v - lo
            m = (dloc >= 0) & (dloc < NLOC)
            eid = (cbase + g * 16) + iota16
            mi = jnp.where(m, 1, 0)
            pos = cur + plsc.cumsum(mi) - mi
            plsc.store_scatter(sele, [pos], eid, mask=m)
            plsc.store_scatter(seld, [pos], dloc, mask=m)
            return cur + plsc.all_reduce_population_count(m)[0]

        cur = lax.fori_loop(0, CH2 // 16, sg, 0)
        sele[pl.ds(cur, 16)] = izero
        seld[pl.ds(cur, 16)] = jnp.full((16,), NLOC, jnp.int32)
        return cur

    def gblk(b, zb, sem):
        pltpu.async_copy(z_hbm.at[sele.at[pl.ds(b * 128, 128)]], zb, sem)

    def wblk(zb, sem):
        pltpu.make_async_copy(z_hbm.at[pl.ds(0, 128)], zb, sem).wait()

    def upd_block(b, zb, ng):
        def grp(g, _):
            dv = seld[pl.ds(g * 16, 16)]
            off = (g - 8 * b) * 16
            for i in range(16):
                d = dv[i]
                for k in range(8):
                    sl = pl.ds(16 * k, 16)
                    acc[d, sl] = jnp.maximum(acc[d, sl], zb[off + i, sl])
            return 0

        lax.fori_loop(8 * b, jnp.minimum(8 * b + 8, ng), grp, 0)

    def proc(cur):
        nb = (cur + 127) // 128
        ng = (cur + 15) // 16

        @pl.when(nb > 0)
        def _():
            gblk(0, zb0, sem_g0)

        def bpair(bb, _):
            b0 = 2 * bb
            b1 = b0 + 1

            @pl.when(b1 < nb)
            def _():
                gblk(b1, zb1, sem_g1)

            wblk(zb0, sem_g0)
            upd_block(b0, zb0, ng)

            @pl.when(b1 + 1 < nb)
            def _():
                gblk(b1 + 1, zb0, sem_g0)

            @pl.when(b1 < nb)
            def _():
                wblk(zb1, sem_g1)
                upd_block(b1, zb1, ng)

            return 0

        lax.fori_loop(0, (nb + 1) // 2, bpair, 0)

    pltpu.async_copy(dst_hbm.at[pl.ds(0, CH2)], dstv0, sem_c0)
    pltpu.async_copy(dst_hbm.at[pl.ds(CH2, CH2)], dstv1, sem_c1)

    def cpair(i, _):
        a = 2 * i
        pltpu.make_async_copy(dst_hbm.at[pl.ds(0, CH2)], dstv0, sem_c0).wait()
        cur = scan(dstv0, a * CH2)

        @pl.when(a + 2 < NCH2)
        def _():
            pltpu.async_copy(dst_hbm.at[pl.ds((a + 2) * CH2, CH2)], dstv0,
                             sem_c0)

        proc(cur)
        pltpu.make_async_copy(dst_hbm.at[pl.ds(0, CH2)], dstv1, sem_c1).wait()
        cur = scan(dstv1, (a + 1) * CH2)

        @pl.when(a + 3 < NCH2)
        def _():
            pltpu.async_copy(dst_hbm.at[pl.ds((a + 3) * CH2, CH2)], dstv1,
                             sem_c1)

        proc(cur)
        return 0

    lax.fori_loop(0, NCH2 // 2, cpair, 0)
    pltpu.sync_copy(acc.at[pl.ds(0, NLOC)], r_hbm.at[pl.ds(lo, NLOC)])


def _sc_pass2(dst, z):
    mesh = plsc.VectorSubcoreMesh(core_axis_name="c", subcore_axis_name="s")
    return pl.kernel(
        _sc2_body,
        out_type=jax.ShapeDtypeStruct((NW * NLOC, D), jnp.float32),
        mesh=mesh,
        compiler_params=pltpu.CompilerParams(needs_layout_passes=False),
        scratch_types=[pltpu.VMEM((CH2,), jnp.int32),
                       pltpu.VMEM((CH2,), jnp.int32),
                       pltpu.VMEM((SELC,), jnp.int32),
                       pltpu.VMEM((SELC,), jnp.int32),
                       pltpu.VMEM((128, D), jnp.float32),
                       pltpu.VMEM((128, D), jnp.float32),
                       pltpu.VMEM((NLOC + 1, D), jnp.float32),
                       pltpu.SemaphoreType.DMA,
                       pltpu.SemaphoreType.DMA,
                       pltpu.SemaphoreType.DMA,
                       pltpu.SemaphoreType.DMA],
    )(dst, z)


def _tc_post_body(r_ref, f_ref, sz_ref, szz_ref, g2, bt2,
                  wu1t, bu1, gu1, btu1, wu2t, bu2, gu2, btu2, out_ref):
    m2 = sz_ref[...] * (1.0 / E)
    v2 = szz_ref[...] * (1.0 / E) - m2 * m2
    inv2 = g2[...] / jnp.sqrt(v2 + EPS)
    r = r_ref[...]
    agg = jnp.maximum((r - m2) * inv2 + bt2[...], 0.0)
    x = jnp.maximum(jnp.dot(agg, wu1t[...], preferred_element_type=jnp.float32)
                    + bu1[...], 0.0)
    x = _bn_train(x, gu1[...], btu1[...])
    x = jnp.maximum(jnp.dot(x, wu2t[...], preferred_element_type=jnp.float32)
                    + bu2[...], 0.0)
    x = _bn_train(x, gu2[...], btu2[...])
    out_ref[...] = x + f_ref[...]


def _tc_post(r, f, sz, szz, g2, bt2, wu1t, bu1, gu1, btu1, wu2t, bu2, gu2,
             btu2):
    return pl.pallas_call(
        _tc_post_body,
        out_shape=jax.ShapeDtypeStruct((N, D), jnp.float32),
    )(r, f, sz, szz, g2, bt2, wu1t, bu1, gu1, btu1, wu2t, bu2, gu2, btu2)


def kernel(input_vertex_features, input_vertex_coordinates, keypoint_indices,
           edges, ao_params, edge_params, update_params):
    f = input_vertex_features
    c = input_vertex_coordinates
    del keypoint_indices

    (wa1, ba1, ga1, bta1), (wa2, ba2, ga2, bta2) = ao_params
    (w1, b1, g1, bt1), (w2, b2, g2, bt2) = edge_params
    (wu1, bu1, gu1, btu1), (wu2, bu2, gu2, btu2) = update_params

    row = lambda v: v.reshape(1, -1)
    src = edges[:, 0]
    dst = edges[:, 1]

    p, q = _tc_pre(f, c, wa1.T, row(ba1), row(ga1), row(bta1),
                   wa2.T, row(ba2), row(ga2), row(bta2),
                   w1[:, :D].T, w1[:, D:].T, row(b1))

    h, sh, sq = _sc_pass1(src, dst, p, q)

    z, sz, szz = _tc_z(h, sh, sq, row(g1), row(bt1), w2.T, row(b2))

    r_full = _sc_pass2(dst, z)

    out = _tc_post(r_full[:N], f, sz, szz, row(g2), row(bt2),
                   wu1.T, row(bu1), row(gu1), row(btu1),
                   wu2.T, row(bu2), row(gu2), row(btu2))
    return out


# speedup vs baseline: 2.9356x; 2.9356x over previous
import jax
import jax.numpy as jnp
from jax import lax
from jax.experimental import pallas as pl
from jax.experimental.pallas import tpu as pltpu
from jax.experimental.pallas import tpu_sc as plsc

N = 10000
E = 320000
D = 128
EPS = 1e-5

NC = 2
NS = 16
NW = NC * NS
E_PER = E // NW
CH1 = 80
NCH1 = E_PER // CH1
NLOC = 320
CH2 = 8000
NCH2 = E // CH2
SELC = ((CH2 + 16 + 127) // 128) * 128
BE = 2000
NBE = E // BE


def _bn_train(x, g, b):
    m = jnp.mean(x, axis=0, keepdims=True)
    v = jnp.mean((x - m) * (x - m), axis=0, keepdims=True)
    return (x - m) / jnp.sqrt(v + EPS) * g + b


def _tc_pre_body(f_ref, c_ref, wa1t, ba1, ga1, bta1, wa2t, ba2, ga2, bta2,
                 w1at, w1bt, b1, p_ref, q_ref):
    f = f_ref[...]
    c = c_ref[...]
    x = jnp.maximum(jnp.dot(f, wa1t[...], preferred_element_type=jnp.float32)
                    + ba1[...], 0.0)
    x = _bn_train(x, ga1[...], bta1[...])
    x = jnp.maximum(jnp.dot(x, wa2t[...], preferred_element_type=jnp.float32)
                    + ba2[...], 0.0)
    off = _bn_train(x, ga2[...], bta2[...])
    c2 = c + off
    p_ref[...] = (jnp.dot(f, w1at[...], preferred_element_type=jnp.float32)
                  + jnp.dot(c, w1bt[...], preferred_element_type=jnp.float32)
                  + b1[...])
    q_ref[...] = jnp.dot(c2, w1bt[...], preferred_element_type=jnp.float32)


def _tc_pre(f, c, wa1t, ba1, ga1, bta1, wa2t, ba2, ga2, bta2, w1at, w1bt, b1):
    return pl.pallas_call(
        _tc_pre_body,
        out_shape=[jax.ShapeDtypeStruct((N, D), jnp.float32),
                   jax.ShapeDtypeStruct((N, D), jnp.float32)],
    )(f, c, wa1t, ba1, ga1, bta1, wa2t, ba2, ga2, bta2, w1at, w1bt, b1)


def _sc1_body(src_hbm, dst_hbm, p_hbm, q_hbm, h_hbm, sh_hbm, sq_hbm,
              srcall, dstall, pv0, qv0, hv0, pv1, qv1, hv1, shv, sqv,
              sem_p0, sem_q0, sem_h0, sem_p1, sem_q1, sem_h1, sem_i):
    wid = lax.axis_index("s") * NC + lax.axis_index("c")
    base0 = wid * E_PER
    zero = jnp.zeros((16,), jnp.float32)
    init = (tuple(zero for _ in range(8)), tuple(zero for _ in range(8)))

    cp_s = pltpu.async_copy(src_hbm.at[pl.ds(base0, E_PER)], srcall, sem_i)
    cp_d = pltpu.async_copy(dst_hbm.at[pl.ds(base0, E_PER)], dstall, sem_i)
    cp_s.wait()
    cp_d.wait()

    def gathers(ci, pv, qv, sp, sq_):
        sl = pl.ds(ci * CH1, CH1)
        pltpu.async_copy(p_hbm.at[srcall.at[sl]], pv, sp)
        pltpu.async_copy(q_hbm.at[dstall.at[sl]], qv, sq_)

    def compute(ci, pv, qv, hv, carry):
        def row(i, cr):
            sh, sq = cr
            nsh = []
            nsq = []
            for k in range(8):
                sl = pl.ds(16 * k, 16)
                h = jnp.maximum(pv[i, sl] - qv[i, sl], 0.0)
                hv[i, sl] = h
                nsh.append(sh[k] + h)
                nsq.append(sq[k] + h * h)
            return (tuple(nsh), tuple(nsq))

        return lax.fori_loop(0, CH1, row, carry)

    def wait_g(pv, qv, sp, sq_):
        pltpu.make_async_copy(p_hbm.at[pl.ds(0, CH1)], pv, sp).wait()
        pltpu.make_async_copy(q_hbm.at[pl.ds(0, CH1)], qv, sq_).wait()

    def wait_h(hv, sh_):
        pltpu.make_async_copy(hv, h_hbm.at[pl.ds(0, CH1)], sh_).wait()

    gathers(0, pv0, qv0, sem_p0, sem_q0)

    def pair(i, carry):
        a = 2 * i
        gathers(a + 1, pv1, qv1, sem_p1, sem_q1)
        wait_g(pv0, qv0, sem_p0, sem_q0)

        @pl.when(i > 0)
        def _():
            wait_h(hv0, sem_h0)

        carry = compute(a, pv0, qv0, hv0, carry)
        pltpu.async_copy(hv0, h_hbm.at[pl.ds(base0 + a * CH1, CH1)], sem_h0)
        gathers(a + 2, pv0, qv0, sem_p0, sem_q0)
        wait_g(pv1, qv1, sem_p1, sem_q1)

        @pl.when(i > 0)
        def _():
            wait_h(hv1, sem_h1)

        carry = compute(a + 1, pv1, qv1, hv1, carry)
        pltpu.async_copy(hv1, h_hbm.at[pl.ds(base0 + (a + 1) * CH1, CH1)],
                         sem_h1)
        return carry

    carry = lax.fori_loop(0, (NCH1 - 1) // 2, pair, init)
    wait_g(pv0, qv0, sem_p0, sem_q0)
    wait_h(hv0, sem_h0)
    sh, sq = compute(NCH1 - 1, pv0, qv0, hv0, carry)
    pltpu.sync_copy(hv0, h_hbm.at[pl.ds(base0 + (NCH1 - 1) * CH1, CH1)])
    wait_h(hv1, sem_h1)

    for k in range(8):
        sl = pl.ds(16 * k, 16)
        shv[0, sl] = sh[k]
        sqv[0, sl] = sq[k]
    pltpu.sync_copy(shv, sh_hbm.at[pl.ds(wid, 1)])
    pltpu.sync_copy(sqv, sq_hbm.at[pl.ds(wid, 1)])


def _sc_pass1(src, dst, p, q):
    mesh = plsc.VectorSubcoreMesh(core_axis_name="c", subcore_axis_name="s")
    return pl.kernel(
        _sc1_body,
        out_type=[jax.ShapeDtypeStruct((E, D), jnp.float32),
                  jax.ShapeDtypeStruct((NW, D), jnp.float32),
                  jax.ShapeDtypeStruct((NW, D), jnp.float32)],
        mesh=mesh,
        compiler_params=pltpu.CompilerParams(needs_layout_passes=False),
        scratch_types=[pltpu.VMEM((E_PER,), jnp.int32),
                       pltpu.VMEM((E_PER,), jnp.int32),
                       pltpu.VMEM((CH1, D), jnp.float32),
                       pltpu.VMEM((CH1, D), jnp.float32),
                       pltpu.VMEM((CH1, D), jnp.float32),
                       pltpu.VMEM((CH1, D), jnp.float32),
                       pltpu.VMEM((CH1, D), jnp.float32),
                       pltpu.VMEM((CH1, D), jnp.float32),
                       pltpu.VMEM((1, D), jnp.float32),
                       pltpu.VMEM((1, D), jnp.float32),
                       pltpu.SemaphoreType.DMA,
                       pltpu.SemaphoreType.DMA,
                       pltpu.SemaphoreType.DMA,
                       pltpu.SemaphoreType.DMA,
                       pltpu.SemaphoreType.DMA,
                       pltpu.SemaphoreType.DMA,
                       pltpu.SemaphoreType.DMA],
    )(src, dst, p, q)


def _tc_z_body(h_ref, sh_ref, sq_ref, g1, bt1, w2t, b2, z_ref, sz_ref, szz_ref):
    i = pl.program_id(0)
    m1 = jnp.sum(sh_ref[...], axis=0, keepdims=True) * (1.0 / E)
    v1 = jnp.sum(sq_ref[...], axis=0, keepdims=True) * (1.0 / E) - m1 * m1
    inv1 = g1[...] / jnp.sqrt(v1 + EPS)
    hn = (h_ref[...] - m1) * inv1 + bt1[...]
    z = jnp.maximum(jnp.dot(hn, w2t[...], preferred_element_type=jnp.float32)
                    + b2[...], 0.0)
    z_ref[...] = z
    bs = jnp.sum(z, axis=0, keepdims=True)
    bss = jnp.sum(z * z, axis=0, keepdims=True)

    @pl.when(i == 0)
    def _():
        sz_ref[...] = bs
        szz_ref[...] = bss

    @pl.when(i > 0)
    def _():
        sz_ref[...] += bs
        szz_ref[...] += bss


def _tc_z(h, sh, sq, g1, bt1, w2t, b2):
    return pl.pallas_call(
        _tc_z_body,
        grid=(NBE,),
        in_specs=[pl.BlockSpec((BE, D), lambda i: (i, 0)),
                  pl.BlockSpec((NW, D), lambda i: (0, 0)),
                  pl.BlockSpec((NW, D), lambda i: (0, 0)),
                  pl.BlockSpec((1, D), lambda i: (0, 0)),
                  pl.BlockSpec((1, D), lambda i: (0, 0)),
                  pl.BlockSpec((D, D), lambda i: (0, 0)),
                  pl.BlockSpec((1, D), lambda i: (0, 0))],
        out_specs=[pl.BlockSpec((BE, D), lambda i: (i, 0)),
                   pl.BlockSpec((1, D), lambda i: (0, 0)),
                   pl.BlockSpec((1, D), lambda i: (0, 0))],
        out_shape=[jax.ShapeDtypeStruct((E, D), jnp.float32),
                   jax.ShapeDtypeStruct((1, D), jnp.float32),
                   jax.ShapeDtypeStruct((1, D), jnp.float32)],
    )(h, sh, sq, g1, bt1, w2t, b2)


def _sc2_body(dst_hbm, z_hbm, r_hbm, dstv0, dstv1, sele, seld, zb0, zb1, acc,
              sem_c0, sem_c1, sem_g0, sem_g1):
    wid = lax.axis_index("s") * NC + lax.axis_index("c")
    lo = wid * NLOC
    zero = jnp.zeros((16,), jnp.float32)
    izero = jnp.zeros((16,), jnp.int32)
    iota16 = lax.iota(jnp.int32, 16)

    def zrow(i, _):
        for k in range(8):
            acc[i, pl.ds(16 * k, 16)] = zero
        return 0

    lax.fori_loop(0, NLOC + 1, zrow, 0)

    def zsel(i, _):
        sele[pl.ds(i * 16, 16)] = izero
        return 0

    lax.fori_loop(0, SELC // 16, zsel, 0)

    def scan(dstv, cbase):
        def sg(g, cur):
            v = dstv[pl.ds(g * 16, 16)]
            dloc = v - lo
            m = (dloc >= 0) & (dloc < NLOC)
            eid = (cbase + g * 16) + iota16
            mi = jnp.where(m, 1, 0)
            pos = cur + plsc.cumsum(mi) - mi
            plsc.store_scatter(sele, [pos], eid, mask=m)
            plsc.store_scatter(seld, [pos], dloc, mask=m)
            return cur + plsc.all_reduce_population_count(m)[0]

        cur = lax.fori_loop(0, CH2 // 16, sg, 0)
        sele[pl.ds(cur, 16)] = izero
        seld[pl.ds(cur, 16)] = jnp.full((16,), NLOC, jnp.int32)
        return cur

    def gblk(g, zb, sem):
        pltpu.async_copy(z_hbm.at[sele.at[pl.ds(g * 16, 16)]], zb, sem)

    def wblk(zb, sem):
        pltpu.make_async_copy(z_hbm.at[pl.ds(0, 16)], zb, sem).wait()

    def upd(g, zb):
        dv = seld[pl.ds(g * 16, 16)]
        for i in range(16):
            d = dv[i]
            for k in range(8):
                sl = pl.ds(16 * k, 16)
                acc[d, sl] = jnp.maximum(acc[d, sl], zb[i, sl])

    def proc(cur):
        ng = (cur + 15) // 16

        @pl.when(ng > 0)
        def _():
            gblk(0, zb0, sem_g0)

        def gpair(bb, _):
            g0 = 2 * bb
            g1 = g0 + 1

            @pl.when(g1 < ng)
            def _():
                gblk(g1, zb1, sem_g1)

            wblk(zb0, sem_g0)
            upd(g0, zb0)

            @pl.when(g1 + 1 < ng)
            def _():
                gblk(g1 + 1, zb0, sem_g0)

            @pl.when(g1 < ng)
            def _():
                wblk(zb1, sem_g1)
                upd(g1, zb1)

            return 0

        lax.fori_loop(0, (ng + 1) // 2, gpair, 0)

    pltpu.async_copy(dst_hbm.at[pl.ds(0, CH2)], dstv0, sem_c0)
    pltpu.async_copy(dst_hbm.at[pl.ds(CH2, CH2)], dstv1, sem_c1)

    def cpair(i, _):
        a = 2 * i
        pltpu.make_async_copy(dst_hbm.at[pl.ds(0, CH2)], dstv0, sem_c0).wait()
        cur = scan(dstv0, a * CH2)

        @pl.when(a + 2 < NCH2)
        def _():
            pltpu.async_copy(dst_hbm.at[pl.ds((a + 2) * CH2, CH2)], dstv0,
                             sem_c0)

        proc(cur)
        pltpu.make_async_copy(dst_hbm.at[pl.ds(0, CH2)], dstv1, sem_c1).wait()
        cur = scan(dstv1, (a + 1) * CH2)

        @pl.when(a + 3 < NCH2)
        def _():
            pltpu.async_copy(dst_hbm.at[pl.ds((a + 3) * CH2, CH2)], dstv1,
                             sem_c1)

        proc(cur)
        return 0

    lax.fori_loop(0, NCH2 // 2, cpair, 0)
    pltpu.sync_copy(acc.at[pl.ds(0, NLOC)], r_hbm.at[pl.ds(lo, NLOC)])


def _sc_pass2(dst, z):
    mesh = plsc.VectorSubcoreMesh(core_axis_name="c", subcore_axis_name="s")
    return pl.kernel(
        _sc2_body,
        out_type=jax.ShapeDtypeStruct((NW * NLOC, D), jnp.float32),
        mesh=mesh,
        compiler_params=pltpu.CompilerParams(needs_layout_passes=False),
        scratch_types=[pltpu.VMEM((CH2,), jnp.int32),
                       pltpu.VMEM((CH2,), jnp.int32),
                       pltpu.VMEM((SELC,), jnp.int32),
                       pltpu.VMEM((SELC,), jnp.int32),
                       pltpu.VMEM((16, D), jnp.float32),
                       pltpu.VMEM((16, D), jnp.float32),
                       pltpu.VMEM((NLOC + 1, D), jnp.float32),
                       pltpu.SemaphoreType.DMA,
                       pltpu.SemaphoreType.DMA,
                       pltpu.SemaphoreType.DMA,
                       pltpu.SemaphoreType.DMA],
    )(dst, z)


def _tc_post_body(r_ref, f_ref, sz_ref, szz_ref, g2, bt2,
                  wu1t, bu1, gu1, btu1, wu2t, bu2, gu2, btu2, out_ref):
    m2 = sz_ref[...] * (1.0 / E)
    v2 = szz_ref[...] * (1.0 / E) - m2 * m2
    inv2 = g2[...] / jnp.sqrt(v2 + EPS)
    r = r_ref[...]
    agg = jnp.maximum((r - m2) * inv2 + bt2[...], 0.0)
    x = jnp.maximum(jnp.dot(agg, wu1t[...], preferred_element_type=jnp.float32)
                    + bu1[...], 0.0)
    x = _bn_train(x, gu1[...], btu1[...])
    x = jnp.maximum(jnp.dot(x, wu2t[...], preferred_element_type=jnp.float32)
                    + bu2[...], 0.0)
    x = _bn_train(x, gu2[...], btu2[...])
    out_ref[...] = x + f_ref[...]


def _tc_post(r, f, sz, szz, g2, bt2, wu1t, bu1, gu1, btu1, wu2t, bu2, gu2,
             btu2):
    return pl.pallas_call(
        _tc_post_body,
        out_shape=jax.ShapeDtypeStruct((N, D), jnp.float32),
    )(r, f, sz, szz, g2, bt2, wu1t, bu1, gu1, btu1, wu2t, bu2, gu2, btu2)


def kernel(input_vertex_features, input_vertex_coordinates, keypoint_indices,
           edges, ao_params, edge_params, update_params):
    f = input_vertex_features
    c = input_vertex_coordinates
    del keypoint_indices

    (wa1, ba1, ga1, bta1), (wa2, ba2, ga2, bta2) = ao_params
    (w1, b1, g1, bt1), (w2, b2, g2, bt2) = edge_params
    (wu1, bu1, gu1, btu1), (wu2, bu2, gu2, btu2) = update_params

    row = lambda v: v.reshape(1, -1)
    src = edges[:, 0]
    dst = edges[:, 1]

    p, q = _tc_pre(f, c, wa1.T, row(ba1), row(ga1), row(bta1),
                   wa2.T, row(ba2), row(ga2), row(bta2),
                   w1[:, :D].T, w1[:, D:].T, row(b1))

    h, sh, sq = _sc_pass1(src, dst, p, q)

    z, sz, szz = _tc_z(h, sh, sq, row(g1), row(bt1), w2.T, row(b2))

    r_full = _sc_pass2(dst, z)

    out = _tc_post(r_full[:N], f, sz, szz, row(g2), row(bt2),
                   wu1.T, row(bu1), row(gu1), row(btu1),
                   wu2.T, row(bu2), row(gu2), row(btu2))
    return out


# speedup vs baseline: 2.9924x; 1.0194x over previous
import jax
import jax.numpy as jnp
from jax import lax
from jax.experimental import pallas as pl
from jax.experimental.pallas import tpu as pltpu
from jax.experimental.pallas import tpu_sc as plsc

N = 10000
E = 320000
D = 128
EPS = 1e-5

NC = 2
NS = 16
NW = NC * NS
E_PER = E // NW
CH1 = 80
NCH1 = E_PER // CH1
NLOC = 320
CH2 = 8000
NCH2 = E // CH2
SELC = ((CH2 + 16 + 127) // 128) * 128
BE = 2000
NBE = E // BE


def _bn_train(x, g, b):
    m = jnp.mean(x, axis=0, keepdims=True)
    v = jnp.mean((x - m) * (x - m), axis=0, keepdims=True)
    return (x - m) / jnp.sqrt(v + EPS) * g + b


def _tc_pre_body(f_ref, c_ref, wa1t, ba1, ga1, bta1, wa2t, ba2, ga2, bta2,
                 w1at, w1bt, b1, p_ref, q_ref):
    f = f_ref[...]
    c = c_ref[...]
    x = jnp.maximum(jnp.dot(f, wa1t[...], preferred_element_type=jnp.float32)
                    + ba1[...], 0.0)
    x = _bn_train(x, ga1[...], bta1[...])
    x = jnp.maximum(jnp.dot(x, wa2t[...], preferred_element_type=jnp.float32)
                    + ba2[...], 0.0)
    off = _bn_train(x, ga2[...], bta2[...])
    c2 = c + off
    p_ref[...] = (jnp.dot(f, w1at[...], preferred_element_type=jnp.float32)
                  + jnp.dot(c, w1bt[...], preferred_element_type=jnp.float32)
                  + b1[...])
    q_ref[...] = jnp.dot(c2, w1bt[...], preferred_element_type=jnp.float32)


def _tc_pre(f, c, wa1t, ba1, ga1, bta1, wa2t, ba2, ga2, bta2, w1at, w1bt, b1):
    return pl.pallas_call(
        _tc_pre_body,
        out_shape=[jax.ShapeDtypeStruct((N, D), jnp.float32),
                   jax.ShapeDtypeStruct((N, D), jnp.float32)],
    )(f, c, wa1t, ba1, ga1, bta1, wa2t, ba2, ga2, bta2, w1at, w1bt, b1)


def _sc1_body(src_hbm, dst_hbm, p_hbm, q_hbm, h_hbm, sh_hbm, sq_hbm,
              srcall, dstall, pv0, qv0, hv0, pv1, qv1, hv1, shv, sqv,
              sem_p0, sem_q0, sem_h0, sem_p1, sem_q1, sem_h1, sem_i):
    wid = lax.axis_index("s") * NC + lax.axis_index("c")
    base0 = wid * E_PER
    zero = jnp.zeros((16,), jnp.float32)
    init = (tuple(zero for _ in range(8)), tuple(zero for _ in range(8)))

    cp_s = pltpu.async_copy(src_hbm.at[pl.ds(base0, E_PER)], srcall, sem_i)
    cp_d = pltpu.async_copy(dst_hbm.at[pl.ds(base0, E_PER)], dstall, sem_i)
    cp_s.wait()
    cp_d.wait()

    def gathers(ci, pv, qv, sp, sq_):
        sl = pl.ds(ci * CH1, CH1)
        pltpu.async_copy(p_hbm.at[srcall.at[sl]], pv, sp)
        pltpu.async_copy(q_hbm.at[dstall.at[sl]], qv, sq_)

    def compute(ci, pv, qv, hv, carry):
        def row(i, cr):
            sh, sq = cr
            nsh = []
            nsq = []
            for k in range(8):
                sl = pl.ds(16 * k, 16)
                h = jnp.maximum(pv[i, sl] - qv[i, sl], 0.0)
                hv[i, sl] = h
                nsh.append(sh[k] + h)
                nsq.append(sq[k] + h * h)
            return (tuple(nsh), tuple(nsq))

        return lax.fori_loop(0, CH1, row, carry)

    def wait_g(pv, qv, sp, sq_):
        pltpu.make_async_copy(p_hbm.at[pl.ds(0, CH1)], pv, sp).wait()
        pltpu.make_async_copy(q_hbm.at[pl.ds(0, CH1)], qv, sq_).wait()

    def wait_h(hv, sh_):
        pltpu.make_async_copy(hv, h_hbm.at[pl.ds(0, CH1)], sh_).wait()

    gathers(0, pv0, qv0, sem_p0, sem_q0)

    def pair(i, carry):
        a = 2 * i
        gathers(a + 1, pv1, qv1, sem_p1, sem_q1)
        wait_g(pv0, qv0, sem_p0, sem_q0)

        @pl.when(i > 0)
        def _():
            wait_h(hv0, sem_h0)

        carry = compute(a, pv0, qv0, hv0, carry)
        pltpu.async_copy(hv0, h_hbm.at[pl.ds(base0 + a * CH1, CH1)], sem_h0)
        gathers(a + 2, pv0, qv0, sem_p0, sem_q0)
        wait_g(pv1, qv1, sem_p1, sem_q1)

        @pl.when(i > 0)
        def _():
            wait_h(hv1, sem_h1)

        carry = compute(a + 1, pv1, qv1, hv1, carry)
        pltpu.async_copy(hv1, h_hbm.at[pl.ds(base0 + (a + 1) * CH1, CH1)],
                         sem_h1)
        return carry

    carry = lax.fori_loop(0, (NCH1 - 1) // 2, pair, init)
    wait_g(pv0, qv0, sem_p0, sem_q0)
    wait_h(hv0, sem_h0)
    sh, sq = compute(NCH1 - 1, pv0, qv0, hv0, carry)
    pltpu.sync_copy(hv0, h_hbm.at[pl.ds(base0 + (NCH1 - 1) * CH1, CH1)])
    wait_h(hv1, sem_h1)

    for k in range(8):
        sl = pl.ds(16 * k, 16)
        shv[0, sl] = sh[k]
        sqv[0, sl] = sq[k]
    pltpu.sync_copy(shv, sh_hbm.at[pl.ds(wid, 1)])
    pltpu.sync_copy(sqv, sq_hbm.at[pl.ds(wid, 1)])


def _sc_pass1(src, dst, p, q):
    mesh = plsc.VectorSubcoreMesh(core_axis_name="c", subcore_axis_name="s")
    return pl.kernel(
        _sc1_body,
        out_type=[jax.ShapeDtypeStruct((E, D), jnp.float32),
                  jax.ShapeDtypeStruct((NW, D), jnp.float32),
                  jax.ShapeDtypeStruct((NW, D), jnp.float32)],
        mesh=mesh,
        compiler_params=pltpu.CompilerParams(needs_layout_passes=False),
        scratch_types=[pltpu.VMEM((E_PER,), jnp.int32),
                       pltpu.VMEM((E_PER,), jnp.int32),
                       pltpu.VMEM((CH1, D), jnp.float32),
                       pltpu.VMEM((CH1, D), jnp.float32),
                       pltpu.VMEM((CH1, D), jnp.float32),
                       pltpu.VMEM((CH1, D), jnp.float32),
                       pltpu.VMEM((CH1, D), jnp.float32),
                       pltpu.VMEM((CH1, D), jnp.float32),
                       pltpu.VMEM((1, D), jnp.float32),
                       pltpu.VMEM((1, D), jnp.float32),
                       pltpu.SemaphoreType.DMA,
                       pltpu.SemaphoreType.DMA,
                       pltpu.SemaphoreType.DMA,
                       pltpu.SemaphoreType.DMA,
                       pltpu.SemaphoreType.DMA,
                       pltpu.SemaphoreType.DMA,
                       pltpu.SemaphoreType.DMA],
    )(src, dst, p, q)


def _tc_z_body(h_ref, sh_ref, sq_ref, g1, bt1, w2t, b2, z_ref, sz_ref, szz_ref):
    i = pl.program_id(0)
    m1 = jnp.sum(sh_ref[...], axis=0, keepdims=True) * (1.0 / E)
    v1 = jnp.sum(sq_ref[...], axis=0, keepdims=True) * (1.0 / E) - m1 * m1
    inv1 = g1[...] / jnp.sqrt(v1 + EPS)
    hn = (h_ref[...] - m1) * inv1 + bt1[...]
    z = jnp.maximum(jnp.dot(hn, w2t[...], preferred_element_type=jnp.float32)
                    + b2[...], 0.0)
    z_ref[...] = z
    bs = jnp.sum(z, axis=0, keepdims=True)
    bss = jnp.sum(z * z, axis=0, keepdims=True)

    @pl.when(i == 0)
    def _():
        sz_ref[...] = bs
        szz_ref[...] = bss

    @pl.when(i > 0)
    def _():
        sz_ref[...] += bs
        szz_ref[...] += bss


def _tc_z(h, sh, sq, g1, bt1, w2t, b2):
    return pl.pallas_call(
        _tc_z_body,
        grid=(NBE,),
        in_specs=[pl.BlockSpec((BE, D), lambda i: (i, 0)),
                  pl.BlockSpec((NW, D), lambda i: (0, 0)),
                  pl.BlockSpec((NW, D), lambda i: (0, 0)),
                  pl.BlockSpec((1, D), lambda i: (0, 0)),
                  pl.BlockSpec((1, D), lambda i: (0, 0)),
                  pl.BlockSpec((D, D), lambda i: (0, 0)),
                  pl.BlockSpec((1, D), lambda i: (0, 0))],
        out_specs=[pl.BlockSpec((BE, D), lambda i: (i, 0)),
                   pl.BlockSpec((1, D), lambda i: (0, 0)),
                   pl.BlockSpec((1, D), lambda i: (0, 0))],
        out_shape=[jax.ShapeDtypeStruct((E, D), jnp.float32),
                   jax.ShapeDtypeStruct((1, D), jnp.float32),
                   jax.ShapeDtypeStruct((1, D), jnp.float32)],
    )(h, sh, sq, g1, bt1, w2t, b2)


def _sc2_body(dst_hbm, z_hbm, r_hbm, dstv0, dstv1, sele, seld, zb0, zb1, acc,
              sem_c0, sem_c1, sem_g0, sem_g1):
    wid = lax.axis_index("s") * NC + lax.axis_index("c")
    lo = wid * NLOC
    zero = jnp.zeros((16,), jnp.float32)
    izero = jnp.zeros((16,), jnp.int32)
    iota16 = lax.iota(jnp.int32, 16)

    def zrow(i, _):
        for k in range(8):
            acc[i, pl.ds(16 * k, 16)] = zero
        return 0

    lax.fori_loop(0, NLOC + 1, zrow, 0)

    def zsel(i, _):
        sele[pl.ds(i * 16, 16)] = izero
        return 0

    lax.fori_loop(0, SELC // 16, zsel, 0)

    def scan(dstv, cbase):
        def sg(g, cur):
            v = dstv[pl.ds(g * 16, 16)]
            dloc = v - lo
            m = (dloc >= 0) & (dloc < NLOC)
            eid = (cbase + g * 16) + iota16
            mi = jnp.where(m, 1, 0)
            cs = plsc.cumsum(mi)
            pos = cur + cs - mi
            plsc.store_scatter(sele, [pos], eid, mask=m)
            plsc.store_scatter(seld, [pos], dloc, mask=m)
            return cur + cs[15]

        cur = lax.fori_loop(0, CH2 // 16, sg, 0)
        sele[pl.ds(cur, 16)] = izero
        seld[pl.ds(cur, 16)] = jnp.full((16,), NLOC, jnp.int32)
        return cur

    def gblk(g, zb, sem):
        pltpu.async_copy(z_hbm.at[sele.at[pl.ds(g * 16, 16)]], zb, sem)

    def wblk(zb, sem):
        pltpu.make_async_copy(z_hbm.at[pl.ds(0, 16)], zb, sem).wait()

    def upd(g, zb):
        dv = seld[pl.ds(g * 16, 16)]
        for i in range(16):
            d = dv[i]
            for k in range(8):
                sl = pl.ds(16 * k, 16)
                acc[d, sl] = jnp.maximum(acc[d, sl], zb[i, sl])

    def proc(cur):
        ng = (cur + 15) // 16

        @pl.when(ng > 0)
        def _():
            gblk(0, zb0, sem_g0)

        def gpair(bb, _):
            g0 = 2 * bb
            g1 = g0 + 1

            @pl.when(g1 < ng)
            def _():
                gblk(g1, zb1, sem_g1)

            wblk(zb0, sem_g0)
            upd(g0, zb0)

            @pl.when(g1 + 1 < ng)
            def _():
                gblk(g1 + 1, zb0, sem_g0)

            @pl.when(g1 < ng)
            def _():
                wblk(zb1, sem_g1)
                upd(g1, zb1)

            return 0

        lax.fori_loop(0, (ng + 1) // 2, gpair, 0)

    pltpu.async_copy(dst_hbm.at[pl.ds(0, CH2)], dstv0, sem_c0)
    pltpu.async_copy(dst_hbm.at[pl.ds(CH2, CH2)], dstv1, sem_c1)

    def cpair(i, _):
        a = 2 * i
        pltpu.make_async_copy(dst_hbm.at[pl.ds(0, CH2)], dstv0, sem_c0).wait()
        cur = scan(dstv0, a * CH2)

        @pl.when(a + 2 < NCH2)
        def _():
            pltpu.async_copy(dst_hbm.at[pl.ds((a + 2) * CH2, CH2)], dstv0,
                             sem_c0)

        proc(cur)
        pltpu.make_async_copy(dst_hbm.at[pl.ds(0, CH2)], dstv1, sem_c1).wait()
        cur = scan(dstv1, (a + 1) * CH2)

        @pl.when(a + 3 < NCH2)
        def _():
            pltpu.async_copy(dst_hbm.at[pl.ds((a + 3) * CH2, CH2)], dstv1,
                             sem_c1)

        proc(cur)
        return 0

    lax.fori_loop(0, NCH2 // 2, cpair, 0)
    pltpu.sync_copy(acc.at[pl.ds(0, NLOC)], r_hbm.at[pl.ds(lo, NLOC)])


def _sc_pass2(dst, z):
    mesh = plsc.VectorSubcoreMesh(core_axis_name="c", subcore_axis_name="s")
    return pl.kernel(
        _sc2_body,
        out_type=jax.ShapeDtypeStruct((NW * NLOC, D), jnp.float32),
        mesh=mesh,
        compiler_params=pltpu.CompilerParams(needs_layout_passes=False),
        scratch_types=[pltpu.VMEM((CH2,), jnp.int32),
                       pltpu.VMEM((CH2,), jnp.int32),
                       pltpu.VMEM((SELC,), jnp.int32),
                       pltpu.VMEM((SELC,), jnp.int32),
                       pltpu.VMEM((16, D), jnp.float32),
                       pltpu.VMEM((16, D), jnp.float32),
                       pltpu.VMEM((NLOC + 1, D), jnp.float32),
                       pltpu.SemaphoreType.DMA,
                       pltpu.SemaphoreType.DMA,
                       pltpu.SemaphoreType.DMA,
                       pltpu.SemaphoreType.DMA],
    )(dst, z)


def _tc_post_body(r_ref, f_ref, sz_ref, szz_ref, g2, bt2,
                  wu1t, bu1, gu1, btu1, wu2t, bu2, gu2, btu2, out_ref):
    m2 = sz_ref[...] * (1.0 / E)
    v2 = szz_ref[...] * (1.0 / E) - m2 * m2
    inv2 = g2[...] / jnp.sqrt(v2 + EPS)
    r = r_ref[...]
    agg = jnp.maximum((r - m2) * inv2 + bt2[...], 0.0)
    x = jnp.maximum(jnp.dot(agg, wu1t[...], preferred_element_type=jnp.float32)
                    + bu1[...], 0.0)
    x = _bn_train(x, gu1[...], btu1[...])
    x = jnp.maximum(jnp.dot(x, wu2t[...], preferred_element_type=jnp.float32)
                    + bu2[...], 0.0)
    x = _bn_train(x, gu2[...], btu2[...])
    out_ref[...] = x + f_ref[...]


def _tc_post(r, f, sz, szz, g2, bt2, wu1t, bu1, gu1, btu1, wu2t, bu2, gu2,
             btu2):
    return pl.pallas_call(
        _tc_post_body,
        out_shape=jax.ShapeDtypeStruct((N, D), jnp.float32),
    )(r, f, sz, szz, g2, bt2, wu1t, bu1, gu1, btu1, wu2t, bu2, gu2, btu2)


def kernel(input_vertex_features, input_vertex_coordinates, keypoint_indices,
           edges, ao_params, edge_params, update_params):
    f = input_vertex_features
    c = input_vertex_coordinates
    del keypoint_indices

    (wa1, ba1, ga1, bta1), (wa2, ba2, ga2, bta2) = ao_params
    (w1, b1, g1, bt1), (w2, b2, g2, bt2) = edge_params
    (wu1, bu1, gu1, btu1), (wu2, bu2, gu2, btu2) = update_params

    row = lambda v: v.reshape(1, -1)
    src = edges[:, 0]
    dst = edges[:, 1]

    p, q = _tc_pre(f, c, wa1.T, row(ba1), row(ga1), row(bta1),
                   wa2.T, row(ba2), row(ga2), row(bta2),
                   w1[:, :D].T, w1[:, D:].T, row(b1))

    h, sh, sq = _sc_pass1(src, dst, p, q)

    z, sz, szz = _tc_z(h, sh, sq, row(g1), row(bt1), w2.T, row(b2))

    r_full = _sc_pass2(dst, z)

    out = _tc_post(r_full[:N], f, sz, szz, row(g2), row(bt2),
                   wu1.T, row(bu1), row(gu1), row(btu1),
                   wu2.T, row(bu2), row(gu2), row(btu2))
    return out
